# baseline (speedup 1.0000x reference)
import functools

import jax
import jax.numpy as jnp
from jax import lax
from jax.experimental import pallas as pl
from jax.experimental.pallas import tpu as pltpu
from jax.experimental.pallas import tpu_sc as plsc

B = 1024
D = 512
K = 100000
TOPK = 16

KT = 2048
NT = 49
KPAD = NT * KT
S = 128
NB = KPAD // S
NBLK = 18
CAND = NBLK * S

BIG = 1e30
IBIG = 2**30

NC, NS = 2, 16
NW = NC * NS


def _dist_body(q_ref, x_ref, d2_ref, m_ref):
    i = pl.program_id(0)
    q = q_ref[...]
    x = x_ref[...]
    qx = lax.dot_general(q, x, (((1,), (1,)), ((), ())),
                         preferred_element_type=jnp.float32)
    q2 = jnp.sum(q * q, axis=1, keepdims=True)
    x2 = jnp.sum(x * x, axis=1)[None, :]
    d2 = q2 - 2.0 * qx + x2
    col = i * KT + lax.broadcasted_iota(jnp.int32, (B, KT), 1)
    d2 = jnp.where(col < K, d2, BIG)
    d2_ref[...] = d2
    m_ref[...] = jnp.min(d2.reshape(B, KT // S, S), axis=2)[None]


def _distances(q, fit_x):
    return pl.pallas_call(
        _dist_body,
        grid=(NT,),
        in_specs=[
            pl.BlockSpec((B, D), lambda i: (0, 0)),
            pl.BlockSpec((KT, D), lambda i: (i, 0)),
        ],
        out_specs=[
            pl.BlockSpec((B, KT), lambda i: (0, i)),
            pl.BlockSpec((1, B, KT // S), lambda i: (i, 0, 0)),
        ],
        out_shape=[
            jax.ShapeDtypeStruct((B, KPAD), jnp.float32),
            jax.ShapeDtypeStruct((NT, B, KT // S), jnp.float32),
        ],
    )(q, fit_x)


NBP = 896


def _blocksel_body(m_ref, blk_ref, flat_ref):
    m = jnp.concatenate(
        [m_ref[...], jnp.full((B, NBP - NB), BIG, jnp.float32)], axis=1)
    colio = lax.broadcasted_iota(jnp.int32, (B, NBP), 1)
    cols = []
    for _ in range(NBLK):
        mn = jnp.min(m, axis=1, keepdims=True)
        bi = jnp.min(jnp.where(m == mn, colio, IBIG), axis=1, keepdims=True)
        cols.append(bi)
        m = jnp.where(colio == bi, BIG, m)
    blk = jnp.concatenate(cols, axis=1)
    rowio = lax.broadcasted_iota(jnp.int32, (B, NBLK), 0)
    blk_ref[...] = blk
    flat_ref[...] = rowio * NB + blk


def _block_select(m):
    return pl.pallas_call(
        _blocksel_body,
        out_shape=[
            jax.ShapeDtypeStruct((B, NBLK), jnp.int32),
            jax.ShapeDtypeStruct((B, NBLK), jnp.int32),
        ],
    )(m)


def _topk_body(c_ref, blk_ref, fidx_ref):
    c = c_ref[...]
    blk = blk_ref[...]
    g = jnp.broadcast_to(blk[:, :, None], (B, NBLK, S)).reshape(B, CAND) * S \
        + (lax.broadcasted_iota(jnp.int32, (B, CAND), 1) % S)
    cols = []
    for _ in range(TOPK):
        mn = jnp.min(c, axis=1, keepdims=True)
        gi = jnp.min(jnp.where(c == mn, g, IBIG), axis=1, keepdims=True)
        cols.append(gi)
        c = jnp.where(g == gi, BIG, c)
    fidx_ref[...] = jnp.concatenate(cols, axis=1)


def _topk_select(cand, blk):
    return pl.pallas_call(
        _topk_body,
        out_shape=jax.ShapeDtypeStruct((B, TOPK), jnp.int32),
    )(cand, blk)


@functools.lru_cache(maxsize=None)
def _sc_gather(nrows, rd, chunk):
    per_w = nrows // NW
    assert per_w % chunk == 0 and per_w % 8 == 0
    mesh = plsc.VectorSubcoreMesh(core_axis_name="c", subcore_axis_name="s")

    @functools.partial(
        pl.kernel,
        mesh=mesh,
        out_type=jax.ShapeDtypeStruct((nrows, rd), jnp.float32),
        scratch_types=[
            pltpu.VMEM((chunk,), jnp.int32),
            pltpu.VMEM((chunk, rd), jnp.float32),
            pltpu.SemaphoreType.DMA,
        ],
    )
    def gath(table_hbm, idx_hbm, out_hbm, idx_v, rows_v, sem):
        wid = lax.axis_index("s") * NC + lax.axis_index("c")
        for c in range(per_w // chunk):
            base = wid * per_w + c * chunk
            pltpu.sync_copy(idx_hbm.at[pl.ds(base, chunk)], idx_v)
            pltpu.async_copy(table_hbm.at[idx_v], rows_v, sem).wait()
            pltpu.sync_copy(rows_v, out_hbm.at[pl.ds(base, chunk)])

    return gath


def _gather_rows(table, idx, chunk):
    nrows, rd = idx.shape[0], table.shape[1]
    return _sc_gather(nrows, rd, chunk)(table, idx)


QB = 64


def _mean_body(n_ref, y_ref):
    y_ref[...] = jnp.sum(n_ref[...], axis=1) * jnp.float32(1.0 / TOPK)


def _mean16(nbrs):
    return pl.pallas_call(
        _mean_body,
        grid=(B // QB,),
        in_specs=[pl.BlockSpec((QB, TOPK, D), lambda i: (i, 0, 0))],
        out_specs=pl.BlockSpec((QB, D), lambda i: (i, 0)),
        out_shape=jax.ShapeDtypeStruct((B, D), jnp.float32),
    )(nbrs)


def kernel(x_enc, fit_X):
    b, nf, seq = x_enc.shape
    q = x_enc.reshape(B, D)
    warm = _gather_rows(fit_X.reshape(K * D // S, S),
                        jnp.arange(NW * 8, dtype=jnp.int32), 8)
    d2, m3 = _distances(q, fit_X)
    m = m3.transpose(1, 0, 2).reshape(B, NB)
    blk, flat = _block_select(m)
    cand = _gather_rows(d2.reshape(B * NB, S), flat.reshape(B * NBLK), 576)
    fidx = _topk_select(cand.reshape(B, CAND), blk)
    nbrs = _gather_rows(fit_X, fidx.reshape(B * TOPK), 128)
    y = _mean16(nbrs.reshape(B, TOPK, D))
    y = y + jnp.minimum(warm[0, 0], 0.0) * 0.0
    return y.reshape(b, nf, seq)


# speedup vs baseline: 2.0164x; 2.0164x over previous
import functools

import jax
import jax.numpy as jnp
from jax import lax
from jax.experimental import pallas as pl
from jax.experimental.pallas import tpu as pltpu
from jax.experimental.pallas import tpu_sc as plsc

B = 1024
D = 512
K = 100000
TOPK = 16

KT = 2048
NT = 49
KPAD = NT * KT
S = 128
NB = KPAD // S
NBLK = 18
CAND = NBLK * S

BIG = 1e30
IBIG = 2**30

NC, NS = 2, 16
NW = NC * NS


def _dist_body(q_ref, x_ref, d2_ref, m_ref):
    i = pl.program_id(0)
    q = q_ref[...]
    x = x_ref[...]
    qx = lax.dot_general(q, x, (((1,), (1,)), ((), ())),
                         preferred_element_type=jnp.float32)
    q2 = jnp.sum(q * q, axis=1, keepdims=True)
    x2 = jnp.sum(x * x, axis=1)[None, :]
    d2 = q2 - 2.0 * qx + x2
    col = i * KT + lax.broadcasted_iota(jnp.int32, (B, KT), 1)
    d2 = jnp.where(col < K, d2, BIG).reshape(B, KT // S, S)
    d2_ref[...] = d2
    m_ref[...] = jnp.min(d2, axis=2)[None]


def _distances(q, fit_x):
    return pl.pallas_call(
        _dist_body,
        grid=(NT,),
        in_specs=[
            pl.BlockSpec((B, D), lambda i: (0, 0)),
            pl.BlockSpec((KT, D), lambda i: (i, 0)),
        ],
        out_specs=[
            pl.BlockSpec((B, KT // S, S), lambda i: (0, i, 0)),
            pl.BlockSpec((1, B, KT // S), lambda i: (i, 0, 0)),
        ],
        out_shape=[
            jax.ShapeDtypeStruct((B, NB, S), jnp.float32),
            jax.ShapeDtypeStruct((NT, B, KT // S), jnp.float32),
        ],
    )(q, fit_x)


NBP = 896


def _blocksel_body(m_ref, blk_ref, flat_ref):
    m = jnp.concatenate(
        [m_ref[...], jnp.full((B, NBP - NB), BIG, jnp.float32)], axis=1)
    colio = lax.broadcasted_iota(jnp.int32, (B, NBP), 1)
    cols = []
    for _ in range(NBLK):
        mn = jnp.min(m, axis=1, keepdims=True)
        bi = jnp.min(jnp.where(m == mn, colio, IBIG), axis=1, keepdims=True)
        cols.append(bi)
        m = jnp.where(colio == bi, BIG, m)
    blk = jnp.concatenate(cols, axis=1)
    rowio = lax.broadcasted_iota(jnp.int32, (B, NBLK), 0)
    blk_ref[...] = blk
    flat_ref[...] = rowio * NB + blk


def _block_select(m):
    return pl.pallas_call(
        _blocksel_body,
        out_shape=[
            jax.ShapeDtypeStruct((B, NBLK), jnp.int32),
            jax.ShapeDtypeStruct((B, NBLK), jnp.int32),
        ],
    )(m)


def _topk_body(c_ref, blk_ref, fidx_ref):
    c = c_ref[...]
    blk = blk_ref[...]
    g = jnp.broadcast_to(blk[:, :, None], (B, NBLK, S)).reshape(B, CAND) * S \
        + (lax.broadcasted_iota(jnp.int32, (B, CAND), 1) % S)
    cols = []
    for _ in range(TOPK):
        mn = jnp.min(c, axis=1, keepdims=True)
        gi = jnp.min(jnp.where(c == mn, g, IBIG), axis=1, keepdims=True)
        cols.append(gi)
        c = jnp.where(g == gi, BIG, c)
    fidx_ref[...] = jnp.concatenate(cols, axis=1)


def _topk_select(cand, blk):
    return pl.pallas_call(
        _topk_body,
        out_shape=jax.ShapeDtypeStruct((B, TOPK), jnp.int32),
    )(cand, blk)


@functools.lru_cache(maxsize=None)
def _sc_gather(nrows, rd, chunk):
    per_w = nrows // NW
    assert per_w % chunk == 0 and per_w % 8 == 0
    mesh = plsc.VectorSubcoreMesh(core_axis_name="c", subcore_axis_name="s")

    @functools.partial(
        pl.kernel,
        mesh=mesh,
        out_type=jax.ShapeDtypeStruct((nrows, rd), jnp.float32),
        scratch_types=[
            pltpu.VMEM((chunk,), jnp.int32),
            pltpu.VMEM((chunk, rd), jnp.float32),
            pltpu.SemaphoreType.DMA,
        ],
    )
    def gath(table_hbm, idx_hbm, out_hbm, idx_v, rows_v, sem):
        wid = lax.axis_index("s") * NC + lax.axis_index("c")
        for c in range(per_w // chunk):
            base = wid * per_w + c * chunk
            pltpu.sync_copy(idx_hbm.at[pl.ds(base, chunk)], idx_v)
            pltpu.async_copy(table_hbm.at[idx_v], rows_v, sem).wait()
            pltpu.sync_copy(rows_v, out_hbm.at[pl.ds(base, chunk)])

    return gath


def _gather_rows(table, idx, chunk):
    nrows, rd = idx.shape[0], table.shape[1]
    return _sc_gather(nrows, rd, chunk)(table, idx)


QB = 64


def _mean_body(n_ref, y_ref):
    y_ref[...] = jnp.sum(n_ref[...], axis=1) * jnp.float32(1.0 / TOPK)


def _mean16(nbrs):
    return pl.pallas_call(
        _mean_body,
        grid=(B // QB,),
        in_specs=[pl.BlockSpec((QB, TOPK, D), lambda i: (i, 0, 0))],
        out_specs=pl.BlockSpec((QB, D), lambda i: (i, 0)),
        out_shape=jax.ShapeDtypeStruct((B, D), jnp.float32),
    )(nbrs)


def kernel(x_enc, fit_X):
    b, nf, seq = x_enc.shape
    q = x_enc.reshape(B, D)
    d2, m3 = _distances(q, fit_X)
    m = m3.transpose(1, 0, 2).reshape(B, NB)
    blk, flat = _block_select(m)
    cand = _gather_rows(d2.reshape(B * NB, S), flat.reshape(B * NBLK), 576)
    fidx = _topk_select(cand.reshape(B, CAND), blk)
    nbrs = _gather_rows(fit_X, fidx.reshape(B * TOPK), 128)
    y = _mean16(nbrs.reshape(B, TOPK, D))
    return y.reshape(b, nf, seq)
